# baseline (speedup 1.0000x reference)
import numpy as np

import jax
import jax.numpy as jnp
from jax import lax
from jax.experimental import pallas as pl
from jax.experimental.pallas import tpu as pltpu
from jax.experimental.pallas import tpu_sc as plsc

NUM_CLASSES = 21
OVERLAP = 0.45
TOP_K = 200
CLS_THRESHOLD = 0.01
CAP = 256
NBITS = 26
BASE_BITS = int(np.float32(CLS_THRESHOLD).view(np.int32))
_NC, _NS, _L = 2, 16, 16


def _thresh_body(sc_ref, thr_ref):
    y = sc_ref[...]
    P = y.shape[0]

    def bit_body(k, off):
        nb = off | jnp.left_shift(jnp.int32(1), NBITS - 1 - k)
        midf = lax.bitcast_convert_type(BASE_BITS + nb, jnp.float32)
        cnt = jnp.sum((y > midf).astype(jnp.float32), axis=1, keepdims=True)
        return jnp.where(cnt >= float(TOP_K), nb, off)

    off = lax.fori_loop(0, NBITS, bit_body, jnp.zeros((P, 1), jnp.int32))
    teff = lax.bitcast_convert_type(BASE_BITS + off, jnp.float32)
    thr_ref[...] = jnp.broadcast_to(jnp.maximum(teff, CLS_THRESHOLD), (P, 16))


def _thresh(scores160):
    P, N = scores160.shape
    return pl.pallas_call(
        _thresh_body,
        out_shape=jax.ShapeDtypeStruct((P, 16), jnp.float32),
    )(scores160)


def _sc_select_body(scores_hbm, thr_hbm, s_hbm, e_hbm,
                    cs_out, csb_out, ceb_out,
                    rowa_v, rowb_v, s_v, e_v, thr_v,
                    cands_v, candi_v, csb_v, ceb_v,
                    sem_in, sem_out):
    cid = lax.axis_index("c")
    sid = lax.axis_index("s")
    wid = sid * _NC + cid
    batch = wid // 4
    r0 = wid * 5
    pltpu.sync_copy(s_hbm.at[batch], s_v)
    pltpu.sync_copy(e_hbm.at[batch], e_v)
    pltpu.sync_copy(thr_hbm.at[pl.ds(r0 * 16, 5 * 16)], thr_v)
    iota16 = lax.iota(jnp.int32, _L)
    neg1 = jnp.full((_L,), -1.0, jnp.float32)
    zeros_i = jnp.zeros((_L,), jnp.int32)
    rows = [rowa_v, rowb_v]
    pltpu.async_copy(scores_hbm.at[r0], rowa_v, sem_in)
    for rr in range(5):
        row_v = rows[rr % 2]
        pltpu.make_async_copy(scores_hbm.at[r0 + rr], row_v, sem_in).wait()
        if rr < 4:
            pltpu.async_copy(scores_hbm.at[r0 + rr + 1],
                             rows[(rr + 1) % 2], sem_in)

        def init_body(j, _, rr=rr):
            cands_v[pl.ds(rr * CAP + j * _L, _L)] = neg1
            candi_v[pl.ds(rr * CAP + j * _L, _L)] = zeros_i
            return 0

        lax.fori_loop(0, CAP // _L, init_body, 0)
        thr = thr_v[pl.ds(rr * 16, _L)]

        def chunk_body(i, base, row_v=row_v, thr=thr, rr=rr):
            v = row_v[pl.ds(i * _L, _L)]
            mask = v > thr
            pop = plsc.all_reduce_population_count(mask)

            @pl.when(jnp.any(mask))
            def _():
                pos = plsc.cumsum(mask.astype(jnp.int32))
                idx = base + pos - 1
                mask2 = mask & (idx < CAP)
                plsc.store_scatter(cands_v, [idx + rr * CAP], v, mask=mask2)
                plsc.store_scatter(candi_v, [idx + rr * CAP],
                                   iota16 + i * _L, mask=mask2)

            return base + pop

        lax.fori_loop(0, 20000 // _L, chunk_body,
                      jnp.zeros((_L,), jnp.int32))

        def gat_body(j, _, rr=rr):
            ii = candi_v[pl.ds(rr * CAP + j * _L, _L)]
            csb_v[pl.ds(rr * CAP + j * _L, _L)] = plsc.load_gather(s_v, [ii])
            ceb_v[pl.ds(rr * CAP + j * _L, _L)] = plsc.load_gather(e_v, [ii])
            return 0

        lax.fori_loop(0, CAP // _L, gat_body, 0)
        pltpu.async_copy(cands_v.at[pl.ds(rr * CAP, CAP)],
                         cs_out.at[r0 + rr], sem_out)
        pltpu.async_copy(csb_v.at[pl.ds(rr * CAP, CAP)],
                         csb_out.at[r0 + rr], sem_out)
        pltpu.async_copy(ceb_v.at[pl.ds(rr * CAP, CAP)],
                         ceb_out.at[r0 + rr], sem_out)
    for rr in range(5):
        pltpu.make_async_copy(cands_v.at[pl.ds(rr * CAP, CAP)],
                              cs_out.at[r0 + rr], sem_out).wait()
        pltpu.make_async_copy(csb_v.at[pl.ds(rr * CAP, CAP)],
                              csb_out.at[r0 + rr], sem_out).wait()
        pltpu.make_async_copy(ceb_v.at[pl.ds(rr * CAP, CAP)],
                              ceb_out.at[r0 + rr], sem_out).wait()


def _sc_select(scores160, thr160, sdec, edec):
    P = scores160.shape[0]
    mesh = plsc.VectorSubcoreMesh(core_axis_name="c", subcore_axis_name="s")
    shp = jax.ShapeDtypeStruct((P, CAP), jnp.float32)
    return pl.kernel(
        _sc_select_body,
        out_type=(shp, shp, shp),
        mesh=mesh,
        compiler_params=pltpu.CompilerParams(needs_layout_passes=False),
        scratch_types=[
            pltpu.VMEM((20000,), jnp.float32),
            pltpu.VMEM((20000,), jnp.float32),
            pltpu.VMEM((20000,), jnp.float32),
            pltpu.VMEM((20000,), jnp.float32),
            pltpu.VMEM((5 * 16,), jnp.float32),
            pltpu.VMEM((5 * CAP,), jnp.float32),
            pltpu.VMEM((5 * CAP,), jnp.int32),
            pltpu.VMEM((5 * CAP,), jnp.float32),
            pltpu.VMEM((5 * CAP,), jnp.float32),
            pltpu.SemaphoreType.DMA,
            pltpu.SemaphoreType.DMA,
        ],
    )(scores160, thr160, sdec, edec)


_RB = 8
_KOUT = 256


def _rank_body(v_ref, s_ref, e_ref, ov_ref, os_ref, oe_ref):
    v = v_ref[...]
    vi = v[:, :, None]
    rank = jnp.zeros((_RB, CAP), jnp.float32)
    for jc in range(CAP // 128):
        vj = v[:, jc * 128:(jc + 1) * 128][:, None, :]
        jidx = jc * 128 + lax.broadcasted_iota(jnp.int32, (_RB, CAP, 128), 2)
        iidx = lax.broadcasted_iota(jnp.int32, (_RB, CAP, 128), 1)
        gt = (vj > vi) | ((vj == vi) & (jidx < iidx))
        rank = rank + jnp.sum(gt.astype(jnp.float32), axis=2)
    sby = s_ref[...]
    eby = e_ref[...]
    for kc in range(_KOUT // 128):
        kk = (kc * 128
              + lax.broadcasted_iota(jnp.int32, (_RB, CAP, 128), 2)
              ).astype(jnp.float32)
        onehot = (rank[:, :, None] == kk).astype(jnp.float32)
        ov_ref[:, kc * 128:(kc + 1) * 128] = jnp.sum(
            v[:, :, None] * onehot, axis=1)
        os_ref[:, kc * 128:(kc + 1) * 128] = jnp.sum(
            sby[:, :, None] * onehot, axis=1)
        oe_ref[:, kc * 128:(kc + 1) * 128] = jnp.sum(
            eby[:, :, None] * onehot, axis=1)


def _rank_place(cs, csb, ceb):
    P = cs.shape[0]
    shp = jax.ShapeDtypeStruct((P, _KOUT), jnp.float32)
    spec_in = pl.BlockSpec((_RB, CAP), lambda i: (i, 0))
    spec_out = pl.BlockSpec((_RB, _KOUT), lambda i: (i, 0))
    return pl.pallas_call(
        _rank_body,
        grid=(P // _RB,),
        in_specs=[spec_in] * 3,
        out_specs=[spec_out] * 3,
        out_shape=[shp, shp, shp],
    )(cs, csb, ceb)


def _nms_body(s_ref, e_ref, v_ref, os_ref, oe_ref, ov_ref, keep_ref):
    s = s_ref[...]
    e = e_ref[...]
    v = v_ref[...]
    K = s.shape[0]
    length = jnp.maximum(e - s, 0.0)
    keep_ref[...] = (v > CLS_THRESHOLD).astype(jnp.float32)
    row = lax.broadcasted_iota(jnp.int32, s.shape, 0)

    def body(i, _):
        si = s_ref[pl.ds(i, 1), :]
        ei = e_ref[pl.ds(i, 1), :]
        li = jnp.maximum(ei - si, 0.0)
        cur = keep_ref[pl.ds(i, 1), :]
        inter = jnp.maximum(jnp.minimum(e, ei) - jnp.maximum(s, si), 0.0)
        union = length + li - inter
        iou = inter / (union + 1e-9)
        supp = ((iou > OVERLAP) & (row > i)).astype(jnp.float32)
        keep_ref[...] = keep_ref[...] * (1.0 - cur * supp)
        return 0

    lax.fori_loop(0, K, body, 0)
    keep = keep_ref[...]
    os_ref[...] = s * keep
    oe_ref[...] = e * keep
    ov_ref[...] = v * keep


def _run_nms(s, e, v):
    K, P = s.shape
    shp = jax.ShapeDtypeStruct((K, P), jnp.float32)
    return pl.pallas_call(
        _nms_body,
        out_shape=(shp, shp, shp),
        scratch_shapes=[pltpu.VMEM((K, P), jnp.float32)],
    )(s, e, v)


def kernel(localizations, classifications, localizations_default):
    B, N, C = classifications.shape
    Cm1 = C - 1
    K = TOP_K
    P = B * Cm1
    scores = jax.nn.softmax(classifications, axis=2)
    center = (localizations_default[:, 0]
              + localizations[..., 0] * 0.1 * localizations_default[:, 1])
    width = localizations_default[:, 1] * jnp.exp(localizations[..., 1] * 0.2)
    sdec2 = center - width / 2.0
    edec2 = center + width / 2.0
    scores160 = jnp.transpose(scores[:, :, 1:], (0, 2, 1)).reshape(P, N)
    thr160 = _thresh(scores160)
    cs, csb, ceb = _sc_select(scores160, thr160.reshape(-1), sdec2, edec2)
    ov, os_, oe_ = _rank_place(cs, csb, ceb)
    v2 = jnp.transpose(ov[:, :K])
    s2 = jnp.transpose(os_[:, :K])
    e2 = jnp.transpose(oe_[:, :K])
    fs, fe, fv = _run_nms(s2, e2, v2)
    out = jnp.stack([fs, fe, fv], axis=-1)
    return jnp.transpose(out, (1, 0, 2)).reshape(B, Cm1, K, 3)


# speedup vs baseline: 1.3517x; 1.3517x over previous
import numpy as np

import jax
import jax.numpy as jnp
from jax import lax
from jax.experimental import pallas as pl
from jax.experimental.pallas import tpu as pltpu
from jax.experimental.pallas import tpu_sc as plsc

NUM_CLASSES = 21
OVERLAP = 0.45
TOP_K = 200
CLS_THRESHOLD = 0.01
CAP = 256
NBITS = 26
BASE_BITS = int(np.float32(CLS_THRESHOLD).view(np.int32))
_NC, _NS, _L = 2, 16, 16


def _thresh_body(sc_ref, thr_ref):
    y = sc_ref[...]
    P = y.shape[0]

    def bit_body(k, off):
        nb = off | jnp.left_shift(jnp.int32(1), NBITS - 1 - k)
        midf = lax.bitcast_convert_type(BASE_BITS + nb, jnp.float32)
        cnt = jnp.sum((y > midf).astype(jnp.float32), axis=1, keepdims=True)
        return jnp.where(cnt >= float(TOP_K), nb, off)

    off = lax.fori_loop(0, NBITS, bit_body, jnp.zeros((P, 1), jnp.int32))
    teff = lax.bitcast_convert_type(BASE_BITS + off, jnp.float32)
    thr_ref[...] = jnp.broadcast_to(jnp.maximum(teff, CLS_THRESHOLD), (P, 16))


def _thresh(scores160):
    P, N = scores160.shape
    return pl.pallas_call(
        _thresh_body,
        out_shape=jax.ShapeDtypeStruct((P, 16), jnp.float32),
    )(scores160)


def _sc_select_body(scores_hbm, thr_hbm, s_hbm, e_hbm,
                    cs_out, csb_out, ceb_out,
                    rowa_v, rowb_v, s_v, e_v, thr_v,
                    cands_v, candi_v, csb_v, ceb_v,
                    sem_in, sem_out):
    cid = lax.axis_index("c")
    sid = lax.axis_index("s")
    wid = sid * _NC + cid
    batch = wid // 4
    r0 = wid * 5
    pltpu.sync_copy(s_hbm.at[batch], s_v)
    pltpu.sync_copy(e_hbm.at[batch], e_v)
    pltpu.sync_copy(thr_hbm.at[pl.ds(r0 * 16, 5 * 16)], thr_v)
    iota16 = lax.iota(jnp.int32, _L)
    neg1 = jnp.full((_L,), -1.0, jnp.float32)
    zeros_i = jnp.zeros((_L,), jnp.int32)
    rows = [rowa_v, rowb_v]
    pltpu.async_copy(scores_hbm.at[r0], rowa_v, sem_in)
    for rr in range(5):
        row_v = rows[rr % 2]
        pltpu.make_async_copy(scores_hbm.at[r0 + rr], row_v, sem_in).wait()
        if rr < 4:
            pltpu.async_copy(scores_hbm.at[r0 + rr + 1],
                             rows[(rr + 1) % 2], sem_in)

        def init_body(j, _, rr=rr):
            cands_v[pl.ds(rr * CAP + j * _L, _L)] = neg1
            candi_v[pl.ds(rr * CAP + j * _L, _L)] = zeros_i
            return 0

        lax.fori_loop(0, CAP // _L, init_body, 0)
        thr = thr_v[pl.ds(rr * 16, _L)]

        def chunk_body(i, base, row_v=row_v, thr=thr, rr=rr):
            b = base
            for u in range(4):
                off = i * (4 * _L) + u * _L
                v = row_v[pl.ds(off, _L)]
                mask = v > thr
                pos = plsc.cumsum(mask.astype(jnp.int32))
                idx = b + pos - 1
                mask2 = mask & (idx < CAP)
                plsc.store_scatter(cands_v, [idx + rr * CAP], v, mask=mask2)
                plsc.store_scatter(candi_v, [idx + rr * CAP],
                                   iota16 + off, mask=mask2)
                b = b + plsc.all_reduce_population_count(mask)
            return b

        nfull = 20000 // (4 * _L)
        base = lax.fori_loop(0, nfull, chunk_body,
                             jnp.zeros((_L,), jnp.int32))
        for u in range(2):
            off = nfull * (4 * _L) + u * _L
            v = row_v[pl.ds(off, _L)]
            mask = v > thr
            pos = plsc.cumsum(mask.astype(jnp.int32))
            idx = base + pos - 1
            mask2 = mask & (idx < CAP)
            plsc.store_scatter(cands_v, [idx + rr * CAP], v, mask=mask2)
            plsc.store_scatter(candi_v, [idx + rr * CAP],
                               iota16 + off, mask=mask2)
            base = base + plsc.all_reduce_population_count(mask)

        def gat_body(j, _, rr=rr):
            ii = candi_v[pl.ds(rr * CAP + j * _L, _L)]
            csb_v[pl.ds(rr * CAP + j * _L, _L)] = plsc.load_gather(s_v, [ii])
            ceb_v[pl.ds(rr * CAP + j * _L, _L)] = plsc.load_gather(e_v, [ii])
            return 0

        lax.fori_loop(0, CAP // _L, gat_body, 0)
        pltpu.async_copy(cands_v.at[pl.ds(rr * CAP, CAP)],
                         cs_out.at[r0 + rr], sem_out)
        pltpu.async_copy(csb_v.at[pl.ds(rr * CAP, CAP)],
                         csb_out.at[r0 + rr], sem_out)
        pltpu.async_copy(ceb_v.at[pl.ds(rr * CAP, CAP)],
                         ceb_out.at[r0 + rr], sem_out)
    for rr in range(5):
        pltpu.make_async_copy(cands_v.at[pl.ds(rr * CAP, CAP)],
                              cs_out.at[r0 + rr], sem_out).wait()
        pltpu.make_async_copy(csb_v.at[pl.ds(rr * CAP, CAP)],
                              csb_out.at[r0 + rr], sem_out).wait()
        pltpu.make_async_copy(ceb_v.at[pl.ds(rr * CAP, CAP)],
                              ceb_out.at[r0 + rr], sem_out).wait()


def _sc_select(scores160, thr160, sdec, edec):
    P = scores160.shape[0]
    mesh = plsc.VectorSubcoreMesh(core_axis_name="c", subcore_axis_name="s")
    shp = jax.ShapeDtypeStruct((P, CAP), jnp.float32)
    return pl.kernel(
        _sc_select_body,
        out_type=(shp, shp, shp),
        mesh=mesh,
        compiler_params=pltpu.CompilerParams(needs_layout_passes=False),
        scratch_types=[
            pltpu.VMEM((20000,), jnp.float32),
            pltpu.VMEM((20000,), jnp.float32),
            pltpu.VMEM((20000,), jnp.float32),
            pltpu.VMEM((20000,), jnp.float32),
            pltpu.VMEM((5 * 16,), jnp.float32),
            pltpu.VMEM((5 * CAP,), jnp.float32),
            pltpu.VMEM((5 * CAP,), jnp.int32),
            pltpu.VMEM((5 * CAP,), jnp.float32),
            pltpu.VMEM((5 * CAP,), jnp.float32),
            pltpu.SemaphoreType.DMA,
            pltpu.SemaphoreType.DMA,
        ],
    )(scores160, thr160, sdec, edec)


_RB = 8
_KOUT = 256


def _rank_body(v_ref, s_ref, e_ref, ov_ref, os_ref, oe_ref):
    v = v_ref[...]
    vi = v[:, :, None]
    rank = jnp.zeros((_RB, CAP), jnp.float32)
    for jc in range(CAP // 128):
        vj = v[:, jc * 128:(jc + 1) * 128][:, None, :]
        jidx = jc * 128 + lax.broadcasted_iota(jnp.int32, (_RB, CAP, 128), 2)
        iidx = lax.broadcasted_iota(jnp.int32, (_RB, CAP, 128), 1)
        gt = (vj > vi) | ((vj == vi) & (jidx < iidx))
        rank = rank + jnp.sum(gt.astype(jnp.float32), axis=2)
    sby = s_ref[...]
    eby = e_ref[...]
    for kc in range(_KOUT // 128):
        kk = (kc * 128
              + lax.broadcasted_iota(jnp.int32, (_RB, CAP, 128), 2)
              ).astype(jnp.float32)
        onehot = (rank[:, :, None] == kk).astype(jnp.float32)
        ov_ref[:, kc * 128:(kc + 1) * 128] = jnp.sum(
            v[:, :, None] * onehot, axis=1)
        os_ref[:, kc * 128:(kc + 1) * 128] = jnp.sum(
            sby[:, :, None] * onehot, axis=1)
        oe_ref[:, kc * 128:(kc + 1) * 128] = jnp.sum(
            eby[:, :, None] * onehot, axis=1)


def _rank_place(cs, csb, ceb):
    P = cs.shape[0]
    shp = jax.ShapeDtypeStruct((P, _KOUT), jnp.float32)
    spec_in = pl.BlockSpec((_RB, CAP), lambda i: (i, 0))
    spec_out = pl.BlockSpec((_RB, _KOUT), lambda i: (i, 0))
    return pl.pallas_call(
        _rank_body,
        grid=(P // _RB,),
        in_specs=[spec_in] * 3,
        out_specs=[spec_out] * 3,
        out_shape=[shp, shp, shp],
    )(cs, csb, ceb)


def _nms_body(s_ref, e_ref, v_ref, os_ref, oe_ref, ov_ref, keep_ref):
    s = s_ref[...]
    e = e_ref[...]
    v = v_ref[...]
    K = s.shape[0]
    length = jnp.maximum(e - s, 0.0)
    keep_ref[...] = (v > CLS_THRESHOLD).astype(jnp.float32)
    row = lax.broadcasted_iota(jnp.int32, s.shape, 0)

    def body(i, _):
        si = s_ref[pl.ds(i, 1), :]
        ei = e_ref[pl.ds(i, 1), :]
        li = jnp.maximum(ei - si, 0.0)
        cur = keep_ref[pl.ds(i, 1), :]
        inter = jnp.maximum(jnp.minimum(e, ei) - jnp.maximum(s, si), 0.0)
        union = length + li - inter
        iou = inter / (union + 1e-9)
        supp = ((iou > OVERLAP) & (row > i)).astype(jnp.float32)
        keep_ref[...] = keep_ref[...] * (1.0 - cur * supp)
        return 0

    lax.fori_loop(0, K, body, 0)
    keep = keep_ref[...]
    os_ref[...] = s * keep
    oe_ref[...] = e * keep
    ov_ref[...] = v * keep


def _run_nms(s, e, v):
    K, P = s.shape
    shp = jax.ShapeDtypeStruct((K, P), jnp.float32)
    return pl.pallas_call(
        _nms_body,
        out_shape=(shp, shp, shp),
        scratch_shapes=[pltpu.VMEM((K, P), jnp.float32)],
    )(s, e, v)


def kernel(localizations, classifications, localizations_default):
    B, N, C = classifications.shape
    Cm1 = C - 1
    K = TOP_K
    P = B * Cm1
    scores = jax.nn.softmax(classifications, axis=2)
    center = (localizations_default[:, 0]
              + localizations[..., 0] * 0.1 * localizations_default[:, 1])
    width = localizations_default[:, 1] * jnp.exp(localizations[..., 1] * 0.2)
    sdec2 = center - width / 2.0
    edec2 = center + width / 2.0
    scores160 = jnp.transpose(scores[:, :, 1:], (0, 2, 1)).reshape(P, N)
    thr160 = _thresh(scores160)
    cs, csb, ceb = _sc_select(scores160, thr160.reshape(-1), sdec2, edec2)
    ov, os_, oe_ = _rank_place(cs, csb, ceb)
    v2 = jnp.transpose(ov[:, :K])
    s2 = jnp.transpose(os_[:, :K])
    e2 = jnp.transpose(oe_[:, :K])
    fs, fe, fv = _run_nms(s2, e2, v2)
    out = jnp.stack([fs, fe, fv], axis=-1)
    return jnp.transpose(out, (1, 0, 2)).reshape(B, Cm1, K, 3)


# speedup vs baseline: 1.4013x; 1.0368x over previous
import numpy as np

import jax
import jax.numpy as jnp
from jax import lax
from jax.experimental import pallas as pl
from jax.experimental.pallas import tpu as pltpu
from jax.experimental.pallas import tpu_sc as plsc

NUM_CLASSES = 21
OVERLAP = 0.45
TOP_K = 200
CLS_THRESHOLD = 0.01
CAP = 256
NBITS = 26
BASE_BITS = int(np.float32(CLS_THRESHOLD).view(np.int32))
_NC, _NS, _L = 2, 16, 16


def _thresh_body(sc_ref, thr_ref):
    y = sc_ref[...]
    P = y.shape[0]

    def bit_body(k, off):
        nb = off | jnp.left_shift(jnp.int32(1), NBITS - 1 - k)
        midf = lax.bitcast_convert_type(BASE_BITS + nb, jnp.float32)
        cnt = jnp.sum((y > midf).astype(jnp.float32), axis=1, keepdims=True)
        return jnp.where(cnt >= float(TOP_K), nb, off)

    off = lax.fori_loop(0, NBITS, bit_body, jnp.zeros((P, 1), jnp.int32))
    teff = lax.bitcast_convert_type(BASE_BITS + off, jnp.float32)
    thr_ref[...] = jnp.broadcast_to(jnp.maximum(teff, CLS_THRESHOLD), (P, 16))


def _thresh(scores160):
    P, N = scores160.shape
    return pl.pallas_call(
        _thresh_body,
        out_shape=jax.ShapeDtypeStruct((P, 16), jnp.float32),
    )(scores160)


def _sc_select_body(scores_hbm, thr_hbm, s_hbm, e_hbm,
                    cs_out, csb_out, ceb_out,
                    rowa_v, rowb_v, s_v, e_v, thr_v,
                    cands_v, candi_v, csb_v, ceb_v,
                    sem_in, sem_out):
    cid = lax.axis_index("c")
    sid = lax.axis_index("s")
    wid = sid * _NC + cid
    batch = wid // 4
    r0 = wid * 5
    pltpu.sync_copy(s_hbm.at[batch], s_v)
    pltpu.sync_copy(e_hbm.at[batch], e_v)
    pltpu.sync_copy(thr_hbm.at[pl.ds(r0 * 16, 5 * 16)], thr_v)
    iota16 = lax.iota(jnp.int32, _L)
    neg1 = jnp.full((_L,), -1.0, jnp.float32)
    zeros_i = jnp.zeros((_L,), jnp.int32)
    rows = [rowa_v, rowb_v]
    pltpu.async_copy(scores_hbm.at[r0], rowa_v, sem_in)
    for rr in range(5):
        row_v = rows[rr % 2]
        pltpu.make_async_copy(scores_hbm.at[r0 + rr], row_v, sem_in).wait()
        if rr < 4:
            pltpu.async_copy(scores_hbm.at[r0 + rr + 1],
                             rows[(rr + 1) % 2], sem_in)

        def init_body(j, _, rr=rr):
            cands_v[pl.ds(rr * CAP + j * _L, _L)] = neg1
            candi_v[pl.ds(rr * CAP + j * _L, _L)] = zeros_i
            return 0

        lax.fori_loop(0, CAP // _L, init_body, 0)
        thr = thr_v[pl.ds(rr * 16, _L)]

        def chunk_body(i, base, row_v=row_v, thr=thr, rr=rr):
            b = base
            for u in range(4):
                off = i * (4 * _L) + u * _L
                v = row_v[pl.ds(off, _L)]
                mask = v > thr
                pos = plsc.cumsum(mask.astype(jnp.int32))
                idx = b + pos - 1
                mask2 = mask & (idx < CAP)
                plsc.store_scatter(cands_v, [idx + rr * CAP], v, mask=mask2)
                plsc.store_scatter(candi_v, [idx + rr * CAP],
                                   iota16 + off, mask=mask2)
                b = b + plsc.all_reduce_population_count(mask)
            return b

        nfull = 20000 // (4 * _L)
        base = lax.fori_loop(0, nfull, chunk_body,
                             jnp.zeros((_L,), jnp.int32))
        for u in range(2):
            off = nfull * (4 * _L) + u * _L
            v = row_v[pl.ds(off, _L)]
            mask = v > thr
            pos = plsc.cumsum(mask.astype(jnp.int32))
            idx = base + pos - 1
            mask2 = mask & (idx < CAP)
            plsc.store_scatter(cands_v, [idx + rr * CAP], v, mask=mask2)
            plsc.store_scatter(candi_v, [idx + rr * CAP],
                               iota16 + off, mask=mask2)
            base = base + plsc.all_reduce_population_count(mask)

        def gat_body(j, _, rr=rr):
            ii = candi_v[pl.ds(rr * CAP + j * _L, _L)]
            csb_v[pl.ds(rr * CAP + j * _L, _L)] = plsc.load_gather(s_v, [ii])
            ceb_v[pl.ds(rr * CAP + j * _L, _L)] = plsc.load_gather(e_v, [ii])
            return 0

        lax.fori_loop(0, CAP // _L, gat_body, 0)
        pltpu.async_copy(cands_v.at[pl.ds(rr * CAP, CAP)],
                         cs_out.at[r0 + rr], sem_out)
        pltpu.async_copy(csb_v.at[pl.ds(rr * CAP, CAP)],
                         csb_out.at[r0 + rr], sem_out)
        pltpu.async_copy(ceb_v.at[pl.ds(rr * CAP, CAP)],
                         ceb_out.at[r0 + rr], sem_out)
    for rr in range(5):
        pltpu.make_async_copy(cands_v.at[pl.ds(rr * CAP, CAP)],
                              cs_out.at[r0 + rr], sem_out).wait()
        pltpu.make_async_copy(csb_v.at[pl.ds(rr * CAP, CAP)],
                              csb_out.at[r0 + rr], sem_out).wait()
        pltpu.make_async_copy(ceb_v.at[pl.ds(rr * CAP, CAP)],
                              ceb_out.at[r0 + rr], sem_out).wait()


def _sc_select(scores160, thr160, sdec, edec):
    P = scores160.shape[0]
    mesh = plsc.VectorSubcoreMesh(core_axis_name="c", subcore_axis_name="s")
    shp = jax.ShapeDtypeStruct((P, CAP), jnp.float32)
    return pl.kernel(
        _sc_select_body,
        out_type=(shp, shp, shp),
        mesh=mesh,
        compiler_params=pltpu.CompilerParams(needs_layout_passes=False),
        scratch_types=[
            pltpu.VMEM((20000,), jnp.float32),
            pltpu.VMEM((20000,), jnp.float32),
            pltpu.VMEM((20000,), jnp.float32),
            pltpu.VMEM((20000,), jnp.float32),
            pltpu.VMEM((5 * 16,), jnp.float32),
            pltpu.VMEM((5 * CAP,), jnp.float32),
            pltpu.VMEM((5 * CAP,), jnp.int32),
            pltpu.VMEM((5 * CAP,), jnp.float32),
            pltpu.VMEM((5 * CAP,), jnp.float32),
            pltpu.SemaphoreType.DMA,
            pltpu.SemaphoreType.DMA,
        ],
    )(scores160, thr160, sdec, edec)


_RB = 8
_KOUT = 256


def _rank_body(v_ref, s_ref, e_ref, ov_ref, os_ref, oe_ref):
    v = v_ref[...]
    vi = v[:, :, None]
    rank = jnp.zeros((_RB, CAP), jnp.float32)
    for jc in range(CAP // 128):
        vj = v[:, jc * 128:(jc + 1) * 128][:, None, :]
        jidx = jc * 128 + lax.broadcasted_iota(jnp.int32, (_RB, CAP, 128), 2)
        iidx = lax.broadcasted_iota(jnp.int32, (_RB, CAP, 128), 1)
        gt = (vj > vi) | ((vj == vi) & (jidx < iidx))
        rank = rank + jnp.sum(gt.astype(jnp.float32), axis=2)
    sby = s_ref[...]
    eby = e_ref[...]
    for kc in range(_KOUT // 128):
        kk = (kc * 128
              + lax.broadcasted_iota(jnp.int32, (_RB, CAP, 128), 2)
              ).astype(jnp.float32)
        onehot = (rank[:, :, None] == kk).astype(jnp.float32)
        ov_ref[:, kc * 128:(kc + 1) * 128] = jnp.sum(
            v[:, :, None] * onehot, axis=1)
        os_ref[:, kc * 128:(kc + 1) * 128] = jnp.sum(
            sby[:, :, None] * onehot, axis=1)
        oe_ref[:, kc * 128:(kc + 1) * 128] = jnp.sum(
            eby[:, :, None] * onehot, axis=1)


def _rank_place(cs, csb, ceb):
    P = cs.shape[0]
    shp = jax.ShapeDtypeStruct((P, _KOUT), jnp.float32)
    spec_in = pl.BlockSpec((_RB, CAP), lambda i: (i, 0))
    spec_out = pl.BlockSpec((_RB, _KOUT), lambda i: (i, 0))
    return pl.pallas_call(
        _rank_body,
        grid=(P // _RB,),
        in_specs=[spec_in] * 3,
        out_specs=[spec_out] * 3,
        out_shape=[shp, shp, shp],
    )(cs, csb, ceb)


def _nms_body(s_ref, e_ref, v_ref, os_ref, oe_ref, ov_ref, keep_ref):
    K, P = s_ref.shape
    keep_ref[...] = (v_ref[...] > CLS_THRESHOLD).astype(jnp.float32)
    BS = 8
    for bi in range(K // BS):
        lo = bi * BS
        rem = K - lo
        s_sub = s_ref[lo:, :]
        e_sub = e_ref[lo:, :]
        l_sub = jnp.maximum(e_sub - s_sub, 0.0)
        row = lo + lax.broadcasted_iota(jnp.int32, (rem, P), 0)

        def body(i, _, lo=lo, s_sub=s_sub, e_sub=e_sub, l_sub=l_sub, row=row):
            ii = lo + i
            si = s_ref[pl.ds(ii, 1), :]
            ei = e_ref[pl.ds(ii, 1), :]
            li = jnp.maximum(ei - si, 0.0)
            cur = keep_ref[pl.ds(ii, 1), :]
            inter = jnp.maximum(
                jnp.minimum(e_sub, ei) - jnp.maximum(s_sub, si), 0.0)
            union = l_sub + li - inter
            iou = inter / (union + 1e-9)
            supp = ((iou > OVERLAP) & (row > ii)).astype(jnp.float32)
            keep_ref[lo:, :] = keep_ref[lo:, :] * (1.0 - cur * supp)
            return 0

        lax.fori_loop(0, BS, body, 0)
    keep = keep_ref[...]
    os_ref[...] = s_ref[...] * keep
    oe_ref[...] = e_ref[...] * keep
    ov_ref[...] = v_ref[...] * keep


def _run_nms(s, e, v):
    K, P = s.shape
    shp = jax.ShapeDtypeStruct((K, P), jnp.float32)
    return pl.pallas_call(
        _nms_body,
        out_shape=(shp, shp, shp),
        scratch_shapes=[pltpu.VMEM((K, P), jnp.float32)],
    )(s, e, v)


def kernel(localizations, classifications, localizations_default):
    B, N, C = classifications.shape
    Cm1 = C - 1
    K = TOP_K
    P = B * Cm1
    scores = jax.nn.softmax(classifications, axis=2)
    center = (localizations_default[:, 0]
              + localizations[..., 0] * 0.1 * localizations_default[:, 1])
    width = localizations_default[:, 1] * jnp.exp(localizations[..., 1] * 0.2)
    sdec2 = center - width / 2.0
    edec2 = center + width / 2.0
    scores160 = jnp.transpose(scores[:, :, 1:], (0, 2, 1)).reshape(P, N)
    thr160 = _thresh(scores160)
    cs, csb, ceb = _sc_select(scores160, thr160.reshape(-1), sdec2, edec2)
    ov, os_, oe_ = _rank_place(cs, csb, ceb)
    v2 = jnp.transpose(ov[:, :K])
    s2 = jnp.transpose(os_[:, :K])
    e2 = jnp.transpose(oe_[:, :K])
    fs, fe, fv = _run_nms(s2, e2, v2)
    out = jnp.stack([fs, fe, fv], axis=-1)
    return jnp.transpose(out, (1, 0, 2)).reshape(B, Cm1, K, 3)
